# Initial kernel scaffold; baseline (speedup 1.0000x reference)
import jax
import jax.numpy as jnp
from jax.experimental import pallas as pl


def kernel(pos_input, pos_item, neg_item, input_emb, item_emb):
    raise NotImplementedError("write your pallas kernel here")


# speedup vs baseline: 2.5343x; 2.5343x over previous
import functools

import jax
import jax.numpy as jnp
from jax import lax
from jax.experimental import pallas as pl
from jax.experimental.pallas import tpu as pltpu
from jax.experimental.pallas import tpu_sc as plsc

B = 16384
D = 64
NNEG = 10
NC = 2
NS = 16
NW = NC * NS
EPW = B // NW
CB = 64
NCH = EPW // CB
NIDXROW = 128
NROWS = CB * NNEG
NJ = NROWS // NIDXROW


def _sc_body(pos_idx, item_idx, neg_idx, input_emb, item_emb,
             pos_out, neg_out,
             idxu_v, idxi_v, idxn_v, u_rows, i_rows, n_rows, pos_v, neg_v,
             sem):
  cid = lax.axis_index("c")
  sid = lax.axis_index("s")
  wid = sid * NC + cid

  iota = lax.iota(jnp.int32, 16)

  def chunk_body(c, carry):
    base_e = (wid * NCH + c) * CB
    pltpu.sync_copy(pos_idx.at[pl.ds(base_e, CB)], idxu_v)
    pltpu.sync_copy(item_idx.at[pl.ds(base_e, CB)], idxi_v)
    pltpu.sync_copy(neg_idx.at[pl.ds(base_e * NNEG, NROWS)], idxn_v)

    cps = [
        pltpu.async_copy(input_emb.at[idxu_v], u_rows, sem),
        pltpu.async_copy(item_emb.at[idxi_v], i_rows, sem),
    ]
    for j in range(NJ):
      cps.append(
          pltpu.async_copy(
              item_emb.at[idxn_v.at[pl.ds(j * NIDXROW, NIDXROW)]],
              n_rows.at[pl.ds(j * NIDXROW, NIDXROW)], sem))
    for cp in cps:
      cp.wait()

    for g in range(CB // 16):
      rowu = iota + g * 16
      rowns = [rowu * NNEG + n for n in range(NNEG)]

      def d_body(d, acc):
        accp, accn = acc
        col = jnp.full((16,), d, dtype=jnp.int32)
        u_d = plsc.load_gather(u_rows, [rowu, col])
        i_d = plsc.load_gather(i_rows, [rowu, col])
        accp = accp + u_d * i_d
        accn = tuple(
            accn[n] + plsc.load_gather(n_rows, [rowns[n], col]) * u_d
            for n in range(NNEG))
        return accp, accn

      z = jnp.zeros((16,), jnp.float32)
      accp, accn = lax.fori_loop(0, D, d_body, (z, (z,) * NNEG))
      pos_v[pl.ds(g * 16, 16)] = accp
      for n in range(NNEG):
        neg_v[n, pl.ds(g * 16, 16)] = accn[n]

    pltpu.sync_copy(pos_v, pos_out.at[pl.ds(base_e, CB)])
    pltpu.sync_copy(neg_v, neg_out.at[wid, c])
    return carry

  lax.fori_loop(0, NCH, chunk_body, 0)


_sc_scores = functools.partial(
    pl.kernel,
    mesh=plsc.VectorSubcoreMesh(
        core_axis_name="c", subcore_axis_name="s",
        num_cores=NC, num_subcores=NS),
    out_type=[
        jax.ShapeDtypeStruct((B,), jnp.float32),
        jax.ShapeDtypeStruct((NW, NCH, NNEG, CB), jnp.float32),
    ],
    scratch_types=[
        pltpu.VMEM((CB,), jnp.int32),
        pltpu.VMEM((CB,), jnp.int32),
        pltpu.VMEM((NROWS,), jnp.int32),
        pltpu.VMEM((CB, D), jnp.float32),
        pltpu.VMEM((CB, D), jnp.float32),
        pltpu.VMEM((NROWS, D), jnp.float32),
        pltpu.VMEM((CB,), jnp.float32),
        pltpu.VMEM((NNEG, CB), jnp.float32),
        pltpu.SemaphoreType.DMA,
    ],
    compiler_params=pltpu.CompilerParams(
        needs_layout_passes=False, use_tc_tiling_on_sc=False),
)(_sc_body)


def _tc_finish(pos_ref, neg_ref, out_ref):
  s = jnp.clip(pos_ref[...], -10.0, 10.0)
  t = jnp.clip(neg_ref[...], -10.0, 10.0)
  total = jnp.sum(jax.nn.softplus(-s)) + jnp.sum(jax.nn.softplus(t))
  out_ref[0, 0] = total * (1.0 / B)


def kernel(pos_input, pos_item, neg_item, input_emb, item_emb):
  pos_idx = pos_input.astype(jnp.int32)
  item_idx = pos_item.astype(jnp.int32)
  neg_idx = neg_item.astype(jnp.int32).reshape(B * NNEG)

  pos_sc, neg_sc = _sc_scores(pos_idx, item_idx, neg_idx, input_emb, item_emb)

  out = pl.pallas_call(
      _tc_finish,
      out_shape=jax.ShapeDtypeStruct((1, 1), jnp.float32),
      in_specs=[
          pl.BlockSpec(memory_space=pltpu.VMEM),
          pl.BlockSpec(memory_space=pltpu.VMEM),
      ],
      out_specs=pl.BlockSpec(memory_space=pltpu.SMEM),
  )(pos_sc.reshape(128, 128), neg_sc.reshape(B * NNEG // 128, 128))
  return out[0, 0]
